# Initial kernel scaffold; baseline (speedup 1.0000x reference)
import jax
import jax.numpy as jnp
from jax.experimental import pallas as pl


def kernel(x, indices):
    raise NotImplementedError("write your pallas kernel here")


# speedup vs baseline: 83.6156x; 83.6156x over previous
import functools

import jax
import jax.numpy as jnp
from jax import lax
from jax.experimental import pallas as pl
from jax.experimental.pallas import tpu as pltpu
from jax.experimental.pallas import tpu_sc as plsc

N, C, H, W = 8, 96, 112, 112
HW = H * W
OUT_HW = (2 * H) * (2 * W)
PLANES = N * C
NUM_WORKERS = 32
PLANES_PER_WORKER = PLANES // NUM_WORKERS
VECS_PER_PLANE = HW // 16
ZERO_VECS = OUT_HW // 16


def _unpool_body(x_hbm, idx_hbm, out_hbm, x_v, idx_v, plane_v):
    cid = lax.axis_index("c")
    sid = lax.axis_index("s")
    wid = sid * 2 + cid

    zeros16 = jnp.zeros((16,), jnp.float32)

    def zero_body(j, carry):
        plane_v[pl.ds(j * 16, 16)] = zeros16
        return carry

    lax.fori_loop(0, ZERO_VECS, zero_body, 0, unroll=8)

    def plane_body(t, carry):
        p = wid * PLANES_PER_WORKER + t
        pltpu.sync_copy(x_hbm.at[p], x_v)
        pltpu.sync_copy(idx_hbm.at[p], idx_v)

        def scat(i, carry):
            iv = idx_v[pl.ds(i * 16, 16)]
            xv = x_v[pl.ds(i * 16, 16)]
            plsc.store_scatter(plane_v, [iv], xv)
            return carry

        lax.fori_loop(0, VECS_PER_PLANE, scat, 0, unroll=8)
        pltpu.sync_copy(plane_v, out_hbm.at[p])

        def unscat(i, carry):
            iv = idx_v[pl.ds(i * 16, 16)]
            plsc.store_scatter(plane_v, [iv], zeros16)
            return carry

        lax.fori_loop(0, VECS_PER_PLANE, unscat, 0, unroll=8)
        return carry

    lax.fori_loop(0, PLANES_PER_WORKER, plane_body, 0)


@jax.jit
def _unpool(x2d, idx2d):
    mesh = plsc.VectorSubcoreMesh(core_axis_name="c", subcore_axis_name="s")
    f = functools.partial(
        pl.kernel,
        out_type=jax.ShapeDtypeStruct((PLANES, OUT_HW), jnp.float32),
        mesh=mesh,
        scratch_types=[
            pltpu.VMEM((HW,), jnp.float32),
            pltpu.VMEM((HW,), jnp.int32),
            pltpu.VMEM((OUT_HW,), jnp.float32),
        ],
        compiler_params=pltpu.CompilerParams(needs_layout_passes=False),
    )(_unpool_body)
    return f(x2d, idx2d)


def kernel(x, indices):
    n, c, h, w = x.shape
    x2d = x.reshape(PLANES, HW)
    idx2d = indices.astype(jnp.int32).reshape(PLANES, HW)
    out = _unpool(x2d, idx2d)
    return out.reshape(n, c, 2 * h, 2 * w)
